# baseline (speedup 1.0000x reference)
import functools

import jax
import jax.numpy as jnp
from jax import lax
from jax.experimental import pallas as pl
from jax.experimental.pallas import tpu as pltpu
from jax.experimental.pallas import tpu_sc as plsc

H = 512
W = 512
C = 8
NPIX = H * W
HALF = 12.8
SCALE = 256.0 / HALF
SHIFT = 255.5
TABLE_ROWS = 2 * H * (W // 2)
PAR_STRIDE = H * (W // 2)

_info = plsc.get_sparse_core_info()
NC, NS = _info.num_cores, _info.num_subcores
NW = NC * NS
ROWS_PER_W = H // NW
PTS_PER_W = NPIX // NW
CHUNK = W
NVEC = CHUNK // 16
NDMA = CHUNK // 128


@functools.partial(
    pl.kernel,
    mesh=plsc.VectorSubcoreMesh(core_axis_name="c", subcore_axis_name="s"),
    out_type=jax.ShapeDtypeStruct((1, C, H, W), jnp.float32),
    compiler_params=pltpu.CompilerParams(
        needs_layout_passes=False, use_tc_tiling_on_sc=False),
    scratch_types=[
        pltpu.VMEM((CHUNK,), jnp.float32),
        pltpu.VMEM((CHUNK,), jnp.float32),
        pltpu.VMEM((CHUNK,), jnp.int32),
        pltpu.VMEM((CHUNK,), jnp.int32),
        pltpu.VMEM((CHUNK,), jnp.float32),
        pltpu.VMEM((CHUNK,), jnp.float32),
        pltpu.VMEM((CHUNK,), jnp.float32),
        pltpu.VMEM((CHUNK,), jnp.float32),
        pltpu.VMEM((CHUNK,), jnp.int32),
        pltpu.VMEM((CHUNK,), jnp.int32),
        pltpu.VMEM((CHUNK, 16), jnp.float32),
        pltpu.VMEM((CHUNK, 16), jnp.float32),
        pltpu.VMEM((C, ROWS_PER_W, W), jnp.float32),
        pltpu.SemaphoreType.DMA,
    ],
)
def _lens_sc(table_hbm, alpha_hbm, out_hbm,
             ax_v, ay_v, ri0_v, ri1_v, w00_v, w01_v, w10_v, w11_v,
             o0_v, o1_v, g0_v, g1_v, outb_v, sem):
    wid = lax.axis_index("s") * NC + lax.axis_index("c")
    base_row = wid * ROWS_PER_W
    base_pt = wid * PTS_PER_W
    lane = lax.iota(jnp.int32, 16)
    step = jnp.float32(2.0 * HALF / (H - 1))

    def chunk_body(c, _):
        row = base_row + c
        pltpu.sync_copy(alpha_hbm.at[pl.ds(row * W, W)], ax_v)
        pltpu.sync_copy(alpha_hbm.at[pl.ds(NPIX + row * W, W)], ay_v)
        ty = jnp.float32(-HALF) + row.astype(jnp.float32) * step

        def p1_body(v, _):
            j0 = v * 16
            tx = (j0 + lane).astype(jnp.float32) * step + jnp.float32(-HALF)
            ax = ax_v[pl.ds(j0, 16)]
            ay = ay_v[pl.ds(j0, 16)]
            fx = (tx - ax) * SCALE + SHIFT
            fy = (ty - ay) * SCALE + SHIFT
            fx = jnp.clip(fx, -16384.0, 16384.0)
            fy = jnp.clip(fy, -16384.0, 16384.0)
            tix = fx.astype(jnp.int32)
            x0 = tix - jnp.where(fx < tix.astype(jnp.float32), 1, 0)
            tiy = fy.astype(jnp.int32)
            y0 = tiy - jnp.where(fy < tiy.astype(jnp.float32), 1, 0)
            wx1 = fx - x0.astype(jnp.float32)
            wy1 = fy - y0.astype(jnp.float32)
            wx0 = 1.0 - wx1
            wy0 = 1.0 - wy1
            wx0 = wx0 * jnp.where((x0 >= 0) & (x0 < W), 1.0, 0.0)
            wx1 = wx1 * jnp.where((x0 >= -1) & (x0 < W - 1), 1.0, 0.0)
            wy0 = wy0 * jnp.where((y0 >= 0) & (y0 < H), 1.0, 0.0)
            wy1 = wy1 * jnp.where((y0 >= -1) & (y0 < H - 1), 1.0, 0.0)
            w00_v[pl.ds(j0, 16)] = wy0 * wx0
            w01_v[pl.ds(j0, 16)] = wy0 * wx1
            w10_v[pl.ds(j0, 16)] = wy1 * wx0
            w11_v[pl.ds(j0, 16)] = wy1 * wx1
            xb = jnp.clip(x0, 0, W - 2)
            o0_v[pl.ds(j0, 16)] = jnp.clip(x0 - xb, 0, 1) * 8
            o1_v[pl.ds(j0, 16)] = jnp.clip(x0 + 1 - xb, 0, 1) * 8
            y0c = jnp.clip(y0, 0, H - 1)
            y1c = jnp.clip(y0 + 1, 0, H - 1)
            tcol = (xb & 1) * PAR_STRIDE + (xb >> 1)
            ri0_v[pl.ds(j0, 16)] = y0c * (W // 2) + tcol
            ri1_v[pl.ds(j0, 16)] = y1c * (W // 2) + tcol
            return _

        lax.fori_loop(0, NVEC, p1_body, None)

        cps = [
            pltpu.async_copy(table_hbm.at[riv.at[pl.ds(i * 128, 128)]],
                             gv.at[pl.ds(i * 128, 128)], sem)
            for riv, gv in ((ri0_v, g0_v), (ri1_v, g1_v))
            for i in range(NDMA)
        ]
        for cp in cps:
            cp.wait()

        def p2_body(v, _):
            j0 = v * 16
            r = j0 + lane
            l0 = o0_v[pl.ds(j0, 16)]
            l1 = o1_v[pl.ds(j0, 16)]
            w00 = w00_v[pl.ds(j0, 16)]
            w01 = w01_v[pl.ds(j0, 16)]
            w10 = w10_v[pl.ds(j0, 16)]
            w11 = w11_v[pl.ds(j0, 16)]
            for ch in range(C):
                v00 = plsc.load_gather(g0_v, [r, l0 + ch])
                v01 = plsc.load_gather(g0_v, [r, l1 + ch])
                v10 = plsc.load_gather(g1_v, [r, l0 + ch])
                v11 = plsc.load_gather(g1_v, [r, l1 + ch])
                acc = w00 * v00 + w01 * v01 + w10 * v10 + w11 * v11
                outb_v[ch, c, pl.ds(j0, 16)] = acc
            return _

        lax.fori_loop(0, NVEC, p2_body, None)
        return _

    lax.fori_loop(0, ROWS_PER_W, chunk_body, None)

    for ch in range(C):
        pltpu.sync_copy(outb_v.at[ch],
                        out_hbm.at[0, ch, pl.ds(base_row, ROWS_PER_W)])


def kernel(source_image, alpha):
    img = source_image[0]
    img_t = jnp.transpose(img, (1, 2, 0))
    t_even = img_t.reshape(PAR_STRIDE, 16)
    shifted = jnp.concatenate(
        [img_t[:, 1:, :], jnp.zeros((H, 1, C), img_t.dtype)], axis=1)
    t_odd = shifted.reshape(PAR_STRIDE, 16)
    table = jnp.concatenate([t_even, t_odd], axis=0)
    return _lens_sc(table, alpha.reshape(2 * NPIX))


# speedup vs baseline: 1.7040x; 1.7040x over previous
import functools

import jax
import jax.numpy as jnp
from jax import lax
from jax.experimental import pallas as pl
from jax.experimental.pallas import tpu as pltpu
from jax.experimental.pallas import tpu_sc as plsc

H = 512
W = 512
C = 8
NPIX = H * W
HALF = 12.8
SCALE = 256.0 / HALF
SHIFT = 255.5
PAR_STRIDE = H * (W // 2)
TROWS = 2 * PAR_STRIDE

_info = plsc.get_sparse_core_info()
NC, NS = _info.num_cores, _info.num_subcores
NW = NC * NS
ROWS_PER_W = H // NW
CHUNK = W
NVEC = CHUNK // 16
NDMA = CHUNK // 128
BY = H // NS
GY = 2


@functools.partial(
    pl.kernel,
    mesh=plsc.VectorSubcoreMesh(core_axis_name="c", subcore_axis_name="s"),
    out_type=(
        jax.ShapeDtypeStruct((1, C, H, W), jnp.float32),
        jax.ShapeDtypeStruct((NC * TROWS, 16), jnp.float32),
    ),
    compiler_params=pltpu.CompilerParams(
        needs_layout_passes=False, use_tc_tiling_on_sc=False),
    scratch_types=[
        pltpu.VMEM((C, GY * W), jnp.float32),
        pltpu.VMEM((GY * W // 2, 16), jnp.float32),
        pltpu.VMEM((GY * W // 2, 16), jnp.float32),
        pltpu.VMEM((CHUNK,), jnp.float32),
        pltpu.VMEM((CHUNK,), jnp.float32),
        pltpu.VMEM((CHUNK,), jnp.int32),
        pltpu.VMEM((CHUNK,), jnp.int32),
        pltpu.VMEM((CHUNK,), jnp.float32),
        pltpu.VMEM((CHUNK,), jnp.float32),
        pltpu.VMEM((CHUNK,), jnp.float32),
        pltpu.VMEM((CHUNK,), jnp.float32),
        pltpu.VMEM((CHUNK,), jnp.int32),
        pltpu.VMEM((CHUNK, 16), jnp.float32),
        pltpu.VMEM((CHUNK, 16), jnp.float32),
        pltpu.VMEM((C, ROWS_PER_W, W), jnp.float32),
        pltpu.SemaphoreType.DMA,
    ],
)
def _lens_sc(img_hbm, alpha_hbm, out_hbm, table_hbm,
             inb_v, eb_v, ob_v, ax_v, ay_v, ri0_v, ri1_v,
             w00_v, w01_v, w10_v, w11_v, o1_v, g0_v, g1_v, outb_v, sem):
    sc = lax.axis_index("c")
    ss = lax.axis_index("s")
    wid = ss * NC + sc
    base_row = wid * ROWS_PER_W
    lane = lax.iota(jnp.int32, 16)
    ch_pat = lane & 7
    px_pat = lane >> 3
    tbase = sc * TROWS

    y_base = ss * BY

    def build_group(g, _):
        y0 = y_base + g * GY
        for ch in range(C):
            pltpu.sync_copy(img_hbm.at[ch, pl.ds(y0 * W, GY * W)],
                            inb_v.at[ch])
        for yy in range(GY):
            def build_row(t, _):
                cidx = yy * W + 2 * t + px_pat
                eb_v[yy * (W // 2) + t, :] = plsc.load_gather(
                    inb_v, [ch_pat, cidx])
                ob_v[yy * (W // 2) + t, :] = plsc.load_gather(
                    inb_v, [ch_pat, jnp.minimum(cidx + 1, GY * W - 1)])
                return _
            lax.fori_loop(0, W // 2, build_row, None, unroll=4)
        pltpu.sync_copy(
            eb_v, table_hbm.at[pl.ds(tbase + y0 * (W // 2), GY * (W // 2))])
        pltpu.sync_copy(
            ob_v, table_hbm.at[pl.ds(tbase + PAR_STRIDE + y0 * (W // 2),
                                     GY * (W // 2))])
        return _

    lax.fori_loop(0, BY // GY, build_group, None)
    plsc.subcore_barrier()

    step = jnp.float32(2.0 * HALF / (H - 1))

    def chunk_body(c, _):
        row = base_row + c
        pltpu.sync_copy(alpha_hbm.at[pl.ds(row * W, W)], ax_v)
        pltpu.sync_copy(alpha_hbm.at[pl.ds(NPIX + row * W, W)], ay_v)
        ty = jnp.float32(-HALF) + row.astype(jnp.float32) * step

        def p1_body(v, _):
            j0 = v * 16
            tx = (j0 + lane).astype(jnp.float32) * step + jnp.float32(-HALF)
            ax = ax_v[pl.ds(j0, 16)]
            ay = ay_v[pl.ds(j0, 16)]
            fx = (tx - ax) * SCALE + SHIFT
            fy = (ty - ay) * SCALE + SHIFT
            fx = jnp.clip(fx, -16384.0, 16384.0)
            fy = jnp.clip(fy, -16384.0, 16384.0)
            tix = fx.astype(jnp.int32)
            x0 = tix - jnp.where(fx < tix.astype(jnp.float32), 1, 0)
            tiy = fy.astype(jnp.int32)
            y0 = tiy - jnp.where(fy < tiy.astype(jnp.float32), 1, 0)
            wx1 = fx - x0.astype(jnp.float32)
            wy1 = fy - y0.astype(jnp.float32)
            wx0 = 1.0 - wx1
            wy0 = 1.0 - wy1
            wx0 = wx0 * jnp.where((x0 >= 0) & (x0 < W), 1.0, 0.0)
            wx1 = wx1 * jnp.where((x0 >= -1) & (x0 < W - 1), 1.0, 0.0)
            wy0 = wy0 * jnp.where((y0 >= 0) & (y0 < H), 1.0, 0.0)
            wy1 = wy1 * jnp.where((y0 >= -1) & (y0 < H - 1), 1.0, 0.0)
            w00_v[pl.ds(j0, 16)] = wy0 * wx0
            w01_v[pl.ds(j0, 16)] = wy0 * wx1
            w10_v[pl.ds(j0, 16)] = wy1 * wx0
            w11_v[pl.ds(j0, 16)] = wy1 * wx1
            o1_v[pl.ds(j0, 16)] = jnp.where(x0 < 0, 0, 8)
            xb = jnp.clip(x0, 0, W - 1)
            y0c = jnp.clip(y0, 0, H - 1)
            y1c = jnp.clip(y0 + 1, 0, H - 1)
            tcol = (xb & 1) * PAR_STRIDE + (xb >> 1) + tbase
            ri0_v[pl.ds(j0, 16)] = y0c * (W // 2) + tcol
            ri1_v[pl.ds(j0, 16)] = y1c * (W // 2) + tcol
            return _

        lax.fori_loop(0, NVEC, p1_body, None)

        cps = [
            pltpu.async_copy(table_hbm.at[riv.at[pl.ds(i * 128, 128)]],
                             gv.at[pl.ds(i * 128, 128)], sem)
            for riv, gv in ((ri0_v, g0_v), (ri1_v, g1_v))
            for i in range(NDMA)
        ]
        for cp in cps:
            cp.wait()

        def p2_body(v, _):
            j0 = v * 16
            r = j0 + lane
            l1 = o1_v[pl.ds(j0, 16)]
            w00 = w00_v[pl.ds(j0, 16)]
            w01 = w01_v[pl.ds(j0, 16)]
            w10 = w10_v[pl.ds(j0, 16)]
            w11 = w11_v[pl.ds(j0, 16)]
            for ch in range(C):
                l0 = lane * 0 + ch
                v00 = plsc.load_gather(g0_v, [r, l0])
                v01 = plsc.load_gather(g0_v, [r, l1 + ch])
                v10 = plsc.load_gather(g1_v, [r, l0])
                v11 = plsc.load_gather(g1_v, [r, l1 + ch])
                acc = w00 * v00 + w01 * v01 + w10 * v10 + w11 * v11
                outb_v[ch, c, pl.ds(j0, 16)] = acc
            return _

        lax.fori_loop(0, NVEC, p2_body, None)
        return _

    lax.fori_loop(0, ROWS_PER_W, chunk_body, None)

    for ch in range(C):
        pltpu.sync_copy(outb_v.at[ch],
                        out_hbm.at[0, ch, pl.ds(base_row, ROWS_PER_W)])


def kernel(source_image, alpha):
    img = source_image.reshape(C, NPIX)
    out, _ = _lens_sc(img, alpha.reshape(2 * NPIX))
    return out


# speedup vs baseline: 2.0695x; 1.2145x over previous
import functools

import jax
import jax.numpy as jnp
from jax import lax
from jax.experimental import pallas as pl
from jax.experimental.pallas import tpu as pltpu
from jax.experimental.pallas import tpu_sc as plsc

H = 512
W = 512
C = 8
NPIX = H * W
HALF = 12.8
SCALE = 256.0 / HALF
SHIFT = 255.5
PAR_STRIDE = H * (W // 2)
TROWS = 2 * PAR_STRIDE

_info = plsc.get_sparse_core_info()
NC, NS = _info.num_cores, _info.num_subcores
NW = NC * NS
ROWS_PER_W = H // NW
NVEC = W // 16
NDMA = W // 128
BY = H // NS
GY = 2
NG = BY // GY


@functools.partial(
    pl.kernel,
    mesh=plsc.VectorSubcoreMesh(core_axis_name="c", subcore_axis_name="s"),
    out_type=(
        jax.ShapeDtypeStruct((1, C, H, W), jnp.float32),
        jax.ShapeDtypeStruct((NC * TROWS, 16), jnp.float32),
    ),
    compiler_params=pltpu.CompilerParams(
        needs_layout_passes=False, use_tc_tiling_on_sc=False),
    scratch_types=[
        pltpu.VMEM((2, C, GY * W), jnp.float32),
        pltpu.VMEM((GY * W // 2, 16), jnp.float32),
        pltpu.VMEM((GY * W // 2, 16), jnp.float32),
        pltpu.VMEM((W,), jnp.float32),
        pltpu.VMEM((W,), jnp.float32),
        pltpu.VMEM((W,), jnp.int32),
        pltpu.VMEM((W,), jnp.int32),
        pltpu.VMEM((W,), jnp.float32),
        pltpu.VMEM((W,), jnp.float32),
        pltpu.VMEM((W,), jnp.float32),
        pltpu.VMEM((W,), jnp.float32),
        pltpu.VMEM((W,), jnp.int32),
        pltpu.VMEM((W, 16), jnp.float32),
        pltpu.VMEM((W, 16), jnp.float32),
        pltpu.VMEM((C, ROWS_PER_W, W), jnp.float32),
        pltpu.SemaphoreType.DMA,
        pltpu.SemaphoreType.DMA,
        pltpu.SemaphoreType.DMA,
    ],
)
def _lens_sc(img_hbm, alpha_hbm, out_hbm, table_hbm,
             inb_v, eb_v, ob_v, ax_v, ay_v, ri0_v, ri1_v,
             w00_v, w01_v, w10_v, w11_v, o1_v, g0_v, g1_v, outb_v,
             sem, sem_i0, sem_i1):
    sc = lax.axis_index("c")
    ss = lax.axis_index("s")
    wid = ss * NC + sc
    base_row = wid * ROWS_PER_W
    lane = lax.iota(jnp.int32, 16)
    ch_pat = lane & 7
    px_pat = lane >> 3
    tbase = sc * TROWS
    sem_i = (sem_i0, sem_i1)

    y_base = ss * BY

    def fire_build(g):
        y0 = y_base + g * GY
        return [
            pltpu.async_copy(img_hbm.at[ch, pl.ds(y0 * W, GY * W)],
                             inb_v.at[g & 1, ch], sem_i[g & 1])
            for ch in range(C)
        ]

    def build_group(g):
        pb = g & 1
        for yy in range(GY):
            def build_row(t, _):
                cidx = yy * W + 2 * t + px_pat
                eb_v[yy * (W // 2) + t, :] = plsc.load_gather(
                    inb_v, [cidx * 0 + pb, ch_pat, cidx])
                ob_v[yy * (W // 2) + t, :] = plsc.load_gather(
                    inb_v, [cidx * 0 + pb, ch_pat,
                            jnp.minimum(cidx + 1, GY * W - 1)])
                return _
            lax.fori_loop(0, W // 2, build_row, None, unroll=4)
        y0 = y_base + g * GY
        pltpu.sync_copy(
            eb_v, table_hbm.at[pl.ds(tbase + y0 * (W // 2), GY * (W // 2))])
        pltpu.sync_copy(
            ob_v, table_hbm.at[pl.ds(tbase + PAR_STRIDE + y0 * (W // 2),
                                     GY * (W // 2))])

    pend_b = fire_build(0)
    for g in range(NG):
        nxt = fire_build(g + 1) if g + 1 < NG else None
        for cp in pend_b:
            cp.wait()
        build_group(g)
        pend_b = nxt
    plsc.subcore_barrier()

    step = jnp.float32(2.0 * HALF / (H - 1))

    def chunk_body(c, _):
        row = base_row + c
        pltpu.sync_copy(alpha_hbm.at[pl.ds(row * W, W)], ax_v)
        pltpu.sync_copy(alpha_hbm.at[pl.ds(NPIX + row * W, W)], ay_v)
        ty = jnp.float32(-HALF) + row.astype(jnp.float32) * step

        def p1_body(v, _):
            j0 = v * 16
            tx = (j0 + lane).astype(jnp.float32) * step + jnp.float32(-HALF)
            ax = ax_v[pl.ds(j0, 16)]
            ay = ay_v[pl.ds(j0, 16)]
            fx = (tx - ax) * SCALE + SHIFT
            fy = (ty - ay) * SCALE + SHIFT
            fx = jnp.clip(fx, -16384.0, 16384.0)
            fy = jnp.clip(fy, -16384.0, 16384.0)
            tix = fx.astype(jnp.int32)
            x0 = tix - jnp.where(fx < tix.astype(jnp.float32), 1, 0)
            tiy = fy.astype(jnp.int32)
            y0 = tiy - jnp.where(fy < tiy.astype(jnp.float32), 1, 0)
            wx1 = fx - x0.astype(jnp.float32)
            wy1 = fy - y0.astype(jnp.float32)
            wx0 = 1.0 - wx1
            wy0 = 1.0 - wy1
            wx0 = wx0 * jnp.where((x0 >= 0) & (x0 < W), 1.0, 0.0)
            wx1 = wx1 * jnp.where((x0 >= -1) & (x0 < W - 1), 1.0, 0.0)
            wy0 = wy0 * jnp.where((y0 >= 0) & (y0 < H), 1.0, 0.0)
            wy1 = wy1 * jnp.where((y0 >= -1) & (y0 < H - 1), 1.0, 0.0)
            w00_v[pl.ds(j0, 16)] = wy0 * wx0
            w01_v[pl.ds(j0, 16)] = wy0 * wx1
            w10_v[pl.ds(j0, 16)] = wy1 * wx0
            w11_v[pl.ds(j0, 16)] = wy1 * wx1
            o1_v[pl.ds(j0, 16)] = jnp.where(x0 < 0, 0, 8)
            xb = jnp.clip(x0, 0, W - 1)
            y0c = jnp.clip(y0, 0, H - 1)
            y1c = jnp.clip(y0 + 1, 0, H - 1)
            tcol = (xb & 1) * PAR_STRIDE + (xb >> 1) + tbase
            ri0_v[pl.ds(j0, 16)] = y0c * (W // 2) + tcol
            ri1_v[pl.ds(j0, 16)] = y1c * (W // 2) + tcol
            return _

        lax.fori_loop(0, NVEC, p1_body, None)

        cps = [
            pltpu.async_copy(table_hbm.at[riv.at[pl.ds(i * 128, 128)]],
                             gv.at[pl.ds(i * 128, 128)], sem)
            for riv, gv in ((ri0_v, g0_v), (ri1_v, g1_v))
            for i in range(NDMA)
        ]
        for cp in cps:
            cp.wait()

        def p2_body(v, _):
            j0 = v * 16
            r = j0 + lane
            l1 = o1_v[pl.ds(j0, 16)]
            w00 = w00_v[pl.ds(j0, 16)]
            w01 = w01_v[pl.ds(j0, 16)]
            w10 = w10_v[pl.ds(j0, 16)]
            w11 = w11_v[pl.ds(j0, 16)]
            for ch in range(C):
                l0 = lane * 0 + ch
                lch = l1 + ch
                v00 = plsc.load_gather(g0_v, [r, l0])
                v01 = plsc.load_gather(g0_v, [r, lch])
                v10 = plsc.load_gather(g1_v, [r, l0])
                v11 = plsc.load_gather(g1_v, [r, lch])
                acc = w00 * v00 + w01 * v01 + w10 * v10 + w11 * v11
                outb_v[ch, c, pl.ds(j0, 16)] = acc
            return _

        lax.fori_loop(0, NVEC, p2_body, None)
        return _

    lax.fori_loop(0, ROWS_PER_W, chunk_body, None)

    for ch in range(C):
        pltpu.sync_copy(outb_v.at[ch],
                        out_hbm.at[0, ch, pl.ds(base_row, ROWS_PER_W)])


def kernel(source_image, alpha):
    img = source_image.reshape(C, NPIX)
    out, _ = _lens_sc(img, alpha.reshape(2 * NPIX))
    return out


# speedup vs baseline: 2.4574x; 1.1874x over previous
import functools

import jax
import jax.numpy as jnp
from jax import lax
from jax.experimental import pallas as pl
from jax.experimental.pallas import tpu as pltpu
from jax.experimental.pallas import tpu_sc as plsc

H = 512
W = 512
C = 8
NPIX = H * W
HALF = 12.8
SCALE = 256.0 / HALF
SHIFT = 255.5
TROWS = H * W

_info = plsc.get_sparse_core_info()
NC, NS = _info.num_cores, _info.num_subcores
NW = NC * NS
ROWS_PER_W = H // NW
NVEC = W // 16
NDMA = W // 128
BY = H // NS

_f32 = jnp.float32
_i32 = jnp.int32


@functools.partial(
    pl.kernel,
    mesh=plsc.VectorSubcoreMesh(core_axis_name="c", subcore_axis_name="s"),
    out_type=(
        jax.ShapeDtypeStruct((1, C, H, W), _f32),
        jax.ShapeDtypeStruct((NC * TROWS, 16), _f32),
    ),
    compiler_params=pltpu.CompilerParams(
        needs_layout_passes=False, use_tc_tiling_on_sc=False),
    scratch_types=[
        pltpu.VMEM((2, C, W), _f32),
        pltpu.VMEM((2, W, 16), _f32),
        pltpu.VMEM((ROWS_PER_W * W,), _f32),
        pltpu.VMEM((ROWS_PER_W * W,), _f32),
        pltpu.VMEM((2, W), _i32),
        pltpu.VMEM((2, W), _i32),
        pltpu.VMEM((2, W), _f32),
        pltpu.VMEM((2, W), _f32),
        pltpu.VMEM((2, W), _f32),
        pltpu.VMEM((2, W), _f32),
        pltpu.VMEM((2, W), _i32),
        pltpu.VMEM((2, W, 16), _f32),
        pltpu.VMEM((2, W, 16), _f32),
        pltpu.VMEM((C, 2, W), _f32),
        pltpu.SemaphoreType.DMA,
        pltpu.SemaphoreType.DMA,
        pltpu.SemaphoreType.DMA,
        pltpu.SemaphoreType.DMA,
        pltpu.SemaphoreType.DMA,
        pltpu.SemaphoreType.DMA,
        pltpu.SemaphoreType.DMA,
        pltpu.SemaphoreType.DMA,
    ],
)
def _lens_sc(img_hbm, alpha_hbm, out_hbm, table_hbm,
             inb_v, ebo_v, ax_v, ay_v, ri0_v, ri1_v,
             w00_v, w01_v, w10_v, w11_v, o1_v, g0_v, g1_v, outr_v,
             sem_a, sem_i0, sem_i1, sem_t0, sem_t1, sem_ga, sem_gb, sem_o):
    sc = lax.axis_index("c")
    ss = lax.axis_index("s")
    wid = ss * NC + sc
    base_row = wid * ROWS_PER_W
    lane = lax.iota(_i32, 16)
    ch_pat = lane & 7
    px_pat = lane >> 3
    tbase = sc * TROWS
    sem_i = (sem_i0, sem_i1)
    sem_t = (sem_t0, sem_t1)

    a_cps = [
        pltpu.async_copy(
            alpha_hbm.at[pl.ds(p * NPIX + base_row * W, ROWS_PER_W * W)],
            av, sem_a)
        for p, av in ((0, ax_v), (1, ay_v))
    ]

    y_base = ss * BY

    def fire_build(g):
        return [
            pltpu.async_copy(img_hbm.at[ch, pl.ds((y_base + g) * W, W)],
                             inb_v.at[g & 1, ch], sem_i[g & 1])
            for ch in range(C)
        ]

    def build_group(g):
        pb = g & 1

        def build_row(rr, _):
            cidx = 2 * (rr & 255) + (rr >> 8) + px_pat
            ebo_v[pb, rr, :] = plsc.load_gather(
                inb_v, [lane * 0 + pb, ch_pat, jnp.minimum(cidx, W - 1)])
            return _

        lax.fori_loop(0, W, build_row, None, unroll=4)
        return pltpu.async_copy(
            ebo_v.at[pb],
            table_hbm.at[pl.ds(tbase + (y_base + g) * W, W)], sem_t[pb])

    pend_b = fire_build(0)
    pend_t = [None, None]
    for g in range(BY):
        nxt = fire_build(g + 1) if g + 1 < BY else None
        for cp in pend_b:
            cp.wait()
        if pend_t[g & 1] is not None:
            pend_t[g & 1].wait()
        pend_t[g & 1] = build_group(g)
        pend_b = nxt
    for pt in pend_t:
        if pt is not None:
            pt.wait()
    plsc.subcore_barrier()

    for cp in a_cps:
        cp.wait()

    step = _f32(2.0 * HALF / (H - 1))

    def p1(c, pb):
        ty = _f32(-HALF) + (base_row + c).astype(_f32) * step

        def p1_body(v, _):
            j0 = v * 16
            tx = (j0 + lane).astype(_f32) * step + _f32(-HALF)
            ax = ax_v[pl.ds(c * W + j0, 16)]
            ay = ay_v[pl.ds(c * W + j0, 16)]
            fx = (tx - ax) * SCALE + SHIFT
            fy = (ty - ay) * SCALE + SHIFT
            fx = jnp.clip(fx, -16384.0, 16384.0)
            fy = jnp.clip(fy, -16384.0, 16384.0)
            tix = fx.astype(_i32)
            x0 = tix - jnp.where(fx < tix.astype(_f32), 1, 0)
            tiy = fy.astype(_i32)
            y0 = tiy - jnp.where(fy < tiy.astype(_f32), 1, 0)
            wx1 = fx - x0.astype(_f32)
            wy1 = fy - y0.astype(_f32)
            wx0 = 1.0 - wx1
            wy0 = 1.0 - wy1
            wx0 = wx0 * jnp.where((x0 >= 0) & (x0 < W), 1.0, 0.0)
            wx1 = wx1 * jnp.where((x0 >= -1) & (x0 < W - 1), 1.0, 0.0)
            wy0 = wy0 * jnp.where((y0 >= 0) & (y0 < H), 1.0, 0.0)
            wy1 = wy1 * jnp.where((y0 >= -1) & (y0 < H - 1), 1.0, 0.0)
            w00_v[pb, pl.ds(j0, 16)] = wy0 * wx0
            w01_v[pb, pl.ds(j0, 16)] = wy0 * wx1
            w10_v[pb, pl.ds(j0, 16)] = wy1 * wx0
            w11_v[pb, pl.ds(j0, 16)] = wy1 * wx1
            o1_v[pb, pl.ds(j0, 16)] = jnp.where(x0 < 0, 0, 8)
            xb = jnp.clip(x0, 0, W - 1)
            y0c = jnp.clip(y0, 0, H - 1)
            y1c = jnp.clip(y0 + 1, 0, H - 1)
            tcol = (xb & 1) * 256 + (xb >> 1) + tbase
            ri0_v[pb, pl.ds(j0, 16)] = y0c * W + tcol
            ri1_v[pb, pl.ds(j0, 16)] = y1c * W + tcol
            return _

        lax.fori_loop(0, NVEC, p1_body, None)

    def fire_gathers(pb, sem_g):
        return [
            pltpu.async_copy(table_hbm.at[riv.at[pb, pl.ds(i * 128, 128)]],
                             gv.at[pb, pl.ds(i * 128, 128)], sem_g)
            for riv, gv in ((ri0_v, g0_v), (ri1_v, g1_v))
            for i in range(NDMA)
        ]

    def drain_gathers(sem_g):
        for i in range(2 * NDMA):
            pltpu.make_async_copy(
                table_hbm.at[ri0_v.at[0, pl.ds((i % NDMA) * 128, 128)]],
                g0_v.at[0, pl.ds((i % NDMA) * 128, 128)], sem_g).wait()

    def drain_out():
        for ch in range(C):
            pltpu.make_async_copy(
                outr_v.at[ch], out_hbm.at[0, ch, pl.ds(0, 2)], sem_o).wait()

    def p2(pb, cc):
        pv = lane * 0 + pb

        def p2_body(v, _):
            j0 = v * 16
            r = j0 + lane
            l1 = o1_v[pb, pl.ds(j0, 16)]
            w00 = w00_v[pb, pl.ds(j0, 16)]
            w01 = w01_v[pb, pl.ds(j0, 16)]
            w10 = w10_v[pb, pl.ds(j0, 16)]
            w11 = w11_v[pb, pl.ds(j0, 16)]
            for ch in range(C):
                l0 = lane * 0 + ch
                lch = l1 + ch
                v00 = plsc.load_gather(g0_v, [pv, r, l0])
                v01 = plsc.load_gather(g0_v, [pv, r, lch])
                v10 = plsc.load_gather(g1_v, [pv, r, l0])
                v11 = plsc.load_gather(g1_v, [pv, r, lch])
                acc = w00 * v00 + w01 * v01 + w10 * v10 + w11 * v11
                outr_v[ch, cc, pl.ds(j0, 16)] = acc
            return _

        lax.fori_loop(0, NVEC, p2_body, None)

    def fire_out(c0):
        return [
            pltpu.async_copy(outr_v.at[ch],
                             out_hbm.at[0, ch, pl.ds(base_row + c0, 2)],
                             sem_o)
            for ch in range(C)
        ]

    fire_out(0)
    p1(0, 0)
    fire_gathers(0, sem_ga)

    def sample_pair(k, _):
        c1 = 2 * k + 1
        c2 = jnp.minimum(c1 + 1, ROWS_PER_W - 1)
        p1(c1, 1)
        fire_gathers(1, sem_gb)
        drain_gathers(sem_ga)
        drain_out()
        p2(0, 0)
        p1(c2, 0)
        fire_gathers(0, sem_ga)
        drain_gathers(sem_gb)
        p2(1, 1)
        fire_out(2 * k)
        return _

    lax.fori_loop(0, ROWS_PER_W // 2, sample_pair, None)

    drain_gathers(sem_ga)
    drain_out()


def kernel(source_image, alpha):
    img = source_image.reshape(C, NPIX)
    out, _ = _lens_sc(img, alpha.reshape(2 * NPIX))
    return out
